# Initial kernel scaffold; baseline (speedup 1.0000x reference)
import jax
import jax.numpy as jnp
from jax.experimental import pallas as pl


def kernel(feature_map, proposals, image_shape, W_fc6, b_fc6, W_fc7, b_fc7, W_cls, b_cls, W_reg, b_reg):
    raise NotImplementedError("write your pallas kernel here")


# speedup vs baseline: 1.4852x; 1.4852x over previous
import functools
import math

import jax
import jax.numpy as jnp
from jax import lax
from jax.experimental import pallas as pl
from jax.experimental.pallas import tpu as pltpu
from jax.experimental.pallas import tpu_sc as plsc

_NUM_CLASSES = 21
_POOL = 7
_STRIDE = 16.0
_SCORE_TH = 0.05
_NMS_TH = 0.5
_TOP_N = 100
_PRE_NMS = 2000
_MIN_SIZE = 1.0
_FEAT_HW = 50
_DW_CLAMP = math.log(1000.0 / 16.0)

_N = 1000
_C = 512
_CELLS = _FEAT_HW * _FEAT_HW
_CELLS_PAD = 2560
_P49 = _POOL * _POOL
_NP = _N * _P49
_NP_PAD = 49152
_H = 4096
_BJ = 512
_NJ = _H // _BJ
_NK = _H // _BJ
_CAND = _N * (_NUM_CLASSES - 1)
_ROWS = 160
_LANES = 128
_BIG = 1 << 30


def _tr_body(x_ref, o_ref):
    o_ref[...] = x_ref[...].T


def _transpose_feat(featp):
    return pl.pallas_call(
        _tr_body,
        grid=(_CELLS_PAD // 256,),
        in_specs=[pl.BlockSpec((_C, 256), lambda i: (0, i))],
        out_specs=pl.BlockSpec((256, _C), lambda i: (i, 0)),
        out_shape=jax.ShapeDtypeStruct((_CELLS_PAD, _C), jnp.float32),
    )(featp)


def _idx_body(p_ref, o_ref):
    pp = p_ref[...]
    x1 = pp[:, 0:1] / _STRIDE
    y1 = pp[:, 1:2] / _STRIDE
    x2 = pp[:, 2:3] / _STRIDE
    y2 = pp[:, 3:4] / _STRIDE
    col = lax.broadcasted_iota(jnp.int32, (_N, _LANES), 1)
    px = (col % _POOL).astype(jnp.float32)
    py = (col // _POOL).astype(jnp.float32)
    gx = (px + 0.5) / _POOL
    gy = (py + 0.5) / _POOL
    sx = x1 + gx * (x2 - x1)
    sy = y1 + gy * (y2 - y1)
    ix = jnp.clip(jnp.floor(sx), 0.0, _FEAT_HW - 1).astype(jnp.int32)
    iy = jnp.clip(jnp.floor(sy), 0.0, _FEAT_HW - 1).astype(jnp.int32)
    flat = iy * _FEAT_HW + ix
    o_ref[...] = jnp.where(col < _P49, flat, 0)


def _roi_idx(proposals):
    return pl.pallas_call(
        _idx_body,
        in_specs=[pl.BlockSpec((_N, 4), lambda: (0, 0))],
        out_specs=pl.BlockSpec((_N, _LANES), lambda: (0, 0)),
        out_shape=jax.ShapeDtypeStruct((_N, _LANES), jnp.int32),
    )(proposals)


_NW = 32
_BPW = _NP_PAD // _NW
_GCH = 64


def _sc_gather(table, idx1d):
    mesh = plsc.VectorSubcoreMesh(core_axis_name="core",
                                  subcore_axis_name="subcore")

    @functools.partial(
        pl.kernel,
        out_type=jax.ShapeDtypeStruct((_NP_PAD, _C), jnp.float32),
        scratch_types=[
            pltpu.VMEM((_BPW,), jnp.int32),
            pltpu.VMEM((_GCH, _C), jnp.float32),
            pltpu.SemaphoreType.DMA,
        ],
        mesh=mesh,
    )
    def k(x_hbm, i_hbm, o_hbm, idx_v, rows_v, sem):
        wid = lax.axis_index("subcore") * 2 + lax.axis_index("core")
        base = wid * _BPW
        pltpu.sync_copy(i_hbm.at[pl.ds(base, _BPW)], idx_v)

        @pl.loop(0, _BPW // _GCH)
        def _(c):
            pltpu.async_copy(
                x_hbm.at[idx_v.at[pl.ds(c * _GCH, _GCH)]], rows_v, sem
            ).wait()
            pltpu.sync_copy(rows_v, o_hbm.at[pl.ds(base + c * _GCH, _GCH)])

    return k(table, idx1d)


def _fc6_body(g_ref, b_ref, w_hbm, o_ref, wbuf, sem):
    p = pl.program_id(0)
    j = pl.program_id(1)
    step = p * _NJ + j
    nsteps = _P49 * _NJ

    @pl.when(step == 0)
    def _():
        pltpu.make_async_copy(w_hbm.at[:, p, pl.ds(j * _BJ, _BJ)],
                              wbuf.at[0], sem.at[0]).start()

    @pl.when(step + 1 < nsteps)
    def _():
        jn = jnp.where(j + 1 < _NJ, j + 1, 0)
        pn = jnp.where(j + 1 < _NJ, p, p + 1)
        pltpu.make_async_copy(w_hbm.at[:, pn, pl.ds(jn * _BJ, _BJ)],
                              wbuf.at[(step + 1) % 2],
                              sem.at[(step + 1) % 2]).start()

    slot = step % 2
    pltpu.make_async_copy(w_hbm.at[:, p, pl.ds(j * _BJ, _BJ)],
                          wbuf.at[slot], sem.at[slot]).wait()
    partial = jnp.dot(g_ref[...], wbuf[slot],
                      preferred_element_type=jnp.float32)
    js = pl.ds(j * _BJ, _BJ)

    @pl.when(p == 0)
    def _():
        o_ref[:, js] = partial

    @pl.when(p > 0)
    def _():
        o_ref[:, js] += partial

    @pl.when(p == _P49 - 1)
    def _():
        o_ref[:, js] = jnp.maximum(o_ref[:, js] + b_ref[0:1, js], 0.0)


def _fc6(G, W6r, b6):
    return pl.pallas_call(
        _fc6_body,
        grid=(_P49, _NJ),
        in_specs=[
            pl.BlockSpec((_N, _C), lambda p, j: (p, 0)),
            pl.BlockSpec((1, _H), lambda p, j: (0, 0)),
            pl.BlockSpec(memory_space=pl.ANY),
        ],
        out_specs=pl.BlockSpec((_N, _H), lambda p, j: (0, 0)),
        out_shape=jax.ShapeDtypeStruct((_N, _H), jnp.float32),
        scratch_shapes=[
            pltpu.VMEM((2, _C, _BJ), jnp.float32),
            pltpu.SemaphoreType.DMA((2,)),
        ],
        compiler_params=pltpu.CompilerParams(
            dimension_semantics=("arbitrary", "arbitrary")),
    )(G, b6, W6r)


def _fc7_body(h_ref, w_ref, b_ref, o_ref):
    j = pl.program_id(0)
    k = pl.program_id(1)
    partial = jnp.dot(h_ref[:, pl.ds(k * _BJ, _BJ)], w_ref[...],
                      preferred_element_type=jnp.float32)
    js = pl.ds(j * _BJ, _BJ)

    @pl.when(k == 0)
    def _():
        o_ref[:, js] = partial

    @pl.when(k > 0)
    def _():
        o_ref[:, js] += partial

    @pl.when(k == _NK - 1)
    def _():
        o_ref[:, js] = jnp.maximum(o_ref[:, js] + b_ref[0:1, js], 0.0)


def _fc7(h6, W7, b7):
    return pl.pallas_call(
        _fc7_body,
        grid=(_NJ, _NK),
        in_specs=[
            pl.BlockSpec((_N, _H), lambda j, k: (0, 0)),
            pl.BlockSpec((_BJ, _BJ), lambda j, k: (k, j)),
            pl.BlockSpec((1, _H), lambda j, k: (0, 0)),
        ],
        out_specs=pl.BlockSpec((_N, _H), lambda j, k: (0, 0)),
        out_shape=jax.ShapeDtypeStruct((_N, _H), jnp.float32),
        compiler_params=pltpu.CompilerParams(
            dimension_semantics=("arbitrary", "arbitrary")),
    )(h6, W7, b7)


def _decode_body(h_ref, w_ref, b_ref, p_ref, img_ref, s_ref,
                 x1_ref, y1_ref, x2_ref, y2_ref):
    logits = jnp.dot(h_ref[...], w_ref[...],
                     preferred_element_type=jnp.float32) + b_ref[0:1, :]
    col = lax.broadcasted_iota(jnp.int32, (_N, _LANES), 1)
    ninf = jnp.float32(-jnp.inf)
    clsmask = col < _NUM_CLASSES
    m = jnp.max(jnp.where(clsmask, logits, ninf), axis=1, keepdims=True)
    e = jnp.exp(logits - m)
    ssum = jnp.sum(jnp.where(clsmask, e, 0.0), axis=1, keepdims=True)
    s_ref[...] = e / ssum

    comp = (col - _NUM_CLASSES) % 4
    std = jnp.where(comp < 2, jnp.float32(0.1), jnp.float32(0.2))
    r0 = logits * std
    a0 = r0
    a1 = jnp.roll(r0, -1, axis=1)
    a2 = jnp.roll(r0, -2, axis=1)
    a3 = jnp.roll(r0, -3, axis=1)
    dw = jnp.minimum(a2, jnp.float32(_DW_CLAMP))
    dh = jnp.minimum(a3, jnp.float32(_DW_CLAMP))

    pp = p_ref[...]
    w = pp[:, 2:3] - pp[:, 0:1]
    h = pp[:, 3:4] - pp[:, 1:2]
    cx = pp[:, 0:1] + 0.5 * w
    cy = pp[:, 1:2] + 0.5 * h
    pcx = a0 * w + cx
    pcy = a1 * h + cy
    pw = jnp.exp(dw) * w
    ph = jnp.exp(dh) * h
    him = img_ref[0, 0]
    wim = img_ref[0, 1]
    x1_ref[...] = jnp.clip(pcx - 0.5 * pw, 0.0, wim)
    y1_ref[...] = jnp.clip(pcy - 0.5 * ph, 0.0, him)
    x2_ref[...] = jnp.clip(pcx + 0.5 * pw, 0.0, wim)
    y2_ref[...] = jnp.clip(pcy + 0.5 * ph, 0.0, him)


def _decode(h7, Wcat, bcat, proposals, imgf):
    outs = [jax.ShapeDtypeStruct((_N, _LANES), jnp.float32)] * 5
    return pl.pallas_call(
        _decode_body,
        in_specs=[
            pl.BlockSpec((_N, _H), lambda: (0, 0)),
            pl.BlockSpec((_H, _LANES), lambda: (0, 0)),
            pl.BlockSpec((1, _LANES), lambda: (0, 0)),
            pl.BlockSpec((_N, 4), lambda: (0, 0)),
            pl.BlockSpec((1, 2), lambda: (0, 0)),
        ],
        out_specs=[pl.BlockSpec((_N, _LANES), lambda: (0, 0))] * 5,
        out_shape=outs,
    )(h7, Wcat, bcat, proposals, imgf)


def _nms_body(s_ref, x1_ref, y1_ref, x2_ref, y2_ref, det_ref):
    ninf = jnp.float32(-jnp.inf)
    s = s_ref[...]
    x1 = x1_ref[...]
    y1 = y1_ref[...]
    x2 = x2_ref[...]
    y2 = y2_ref[...]
    fi = (lax.broadcasted_iota(jnp.int32, (_ROWS, _LANES), 0) * _LANES
          + lax.broadcasted_iota(jnp.int32, (_ROWS, _LANES), 1))

    valid = (s > _SCORE_TH) & (x2 - x1 >= _MIN_SIZE) & (y2 - y1 >= _MIN_SIZE)
    msc = jnp.where(valid, s, ninf)
    key = lax.bitcast_convert_type(msc, jnp.int32)

    def bs_val(_, lh):
        lo, hi = lh
        mid = lo + (hi - lo + 1) // 2
        cnt = jnp.sum((key >= mid).astype(jnp.int32))
        ok = cnt >= _PRE_NMS
        return (jnp.where(ok, mid, lo), jnp.where(ok, hi, mid - 1))

    lo0 = jnp.int32(1)
    hi0 = jnp.int32(0x40000000)
    kth, _ = lax.fori_loop(0, 31, bs_val, (lo0, hi0))

    k1 = jnp.sum((key > kth).astype(jnp.int32))
    need = _PRE_NMS - k1
    ties = key == kth

    def bs_idx(_, lh):
        lo, hi = lh
        mid = (lo + hi) // 2
        c = jnp.sum((ties & (fi <= mid)).astype(jnp.int32))
        ok = c >= need
        return (jnp.where(ok, lo, mid + 1), jnp.where(ok, mid, hi))

    mstar, _ = lax.fori_loop(0, 16, bs_idx,
                             (jnp.int32(0), jnp.int32(_ROWS * _LANES - 1)))

    sel = (key > kth) | (ties & (fi <= mstar))
    sc0 = jnp.where(sel, msc, ninf)

    lab_all = (fi % (_NUM_CLASSES - 1) + 1).astype(jnp.float32)
    offv = lab_all * 1000.0
    ox1 = x1 + offv
    oy1 = y1 + offv
    ox2 = x2 + offv
    oy2 = y2 + offv
    area = (ox2 - ox1) * (oy2 - oy1)

    m0 = jnp.max(sc0)
    i_fb = jnp.min(jnp.where(sc0 == m0, fi, _BIG))
    i_fb = jnp.minimum(i_fb, jnp.int32(_CAND - 1))

    zrow = jnp.zeros((1, _LANES), jnp.float32)
    ciota = lax.broadcasted_iota(jnp.int32, (1, _LANES), 1)

    def body(k, carry):
        sc, ax1, ay1, ax2, ay2, alab, asc = carry
        mk = jnp.max(sc)
        i = jnp.min(jnp.where(sc == mk, fi, _BIG))
        i = jnp.where(mk == ninf, i_fb, i)
        pick = fi == i
        bx1 = jnp.sum(jnp.where(pick, x1, 0.0))
        by1 = jnp.sum(jnp.where(pick, y1, 0.0))
        bx2 = jnp.sum(jnp.where(pick, x2, 0.0))
        by2 = jnp.sum(jnp.where(pick, y2, 0.0))
        labi = i % (_NUM_CLASSES - 1) + 1
        labf = labi.astype(jnp.float32)
        off = labf * 1000.0
        b0 = bx1 + off
        b1 = by1 + off
        b2 = bx2 + off
        b3 = by2 + off
        a_b = (b2 - b0) * (b3 - b1)
        xx1 = jnp.maximum(b0, ox1)
        yy1 = jnp.maximum(b1, oy1)
        xx2 = jnp.minimum(b2, ox2)
        yy2 = jnp.minimum(b3, oy2)
        inter = jnp.maximum(xx2 - xx1, 0.0) * jnp.maximum(yy2 - yy1, 0.0)
        iou = inter / (a_b + area - inter)
        sc = jnp.where(iou > _NMS_TH, ninf, sc)
        sc = jnp.where(pick, ninf, sc)
        cm = ciota == k
        ax1 = jnp.where(cm, bx1, ax1)
        ay1 = jnp.where(cm, by1, ay1)
        ax2 = jnp.where(cm, bx2, ax2)
        ay2 = jnp.where(cm, by2, ay2)
        alab = jnp.where(cm, labf, alab)
        asc = jnp.where(cm, mk, asc)
        return sc, ax1, ay1, ax2, ay2, alab, asc

    init = (sc0, zrow, zrow, zrow, zrow, zrow, zrow)
    _, ax1, ay1, ax2, ay2, alab, asc = lax.fori_loop(0, _TOP_N, body, init)
    det_ref[...] = jnp.concatenate(
        [ax1, ay1, ax2, ay2, alab, asc, zrow, zrow], axis=0)


def _nms(s, x1, y1, x2, y2):
    return pl.pallas_call(
        _nms_body,
        in_specs=[pl.BlockSpec((_ROWS, _LANES), lambda: (0, 0))] * 5,
        out_specs=pl.BlockSpec((8, _LANES), lambda: (0, 0)),
        out_shape=jax.ShapeDtypeStruct((8, _LANES), jnp.float32),
    )(s, x1, y1, x2, y2)


def kernel(feature_map, proposals, image_shape, W_fc6, b_fc6, W_fc7, b_fc7,
           W_cls, b_cls, W_reg, b_reg):
    feat = feature_map[0].reshape(_C, _CELLS)
    featp = jnp.pad(feat, ((0, 0), (0, _CELLS_PAD - _CELLS)))
    table = _transpose_feat(featp)

    flat = _roi_idx(proposals)
    idxf = flat[:, :_P49].T.reshape(_NP)
    idxp = jnp.pad(idxf, (0, _NP_PAD - _NP))
    G = _sc_gather(table, idxp)

    W6r = W_fc6.reshape(_C, _P49, _H)
    h6 = _fc6(G, W6r, b_fc6.reshape(1, _H))
    h7 = _fc7(h6, W_fc7, b_fc7.reshape(1, _H))

    npad = _LANES - (_NUM_CLASSES + 4 * _NUM_CLASSES)
    Wcat = jnp.pad(jnp.concatenate([W_cls, W_reg], axis=1),
                   ((0, 0), (0, npad)))
    bcat = jnp.pad(jnp.concatenate([b_cls, b_reg]), (0, npad)).reshape(1, _LANES)
    imgf = image_shape.astype(jnp.float32).reshape(1, 2)
    probs, bx1, by1, bx2, by2 = _decode(h7, Wcat, bcat, proposals, imgf)

    c0 = _NUM_CLASSES + 4
    c1 = _NUM_CLASSES + 4 * _NUM_CLASSES

    def tiles(a, pad_val):
        flat20 = a.reshape(_CAND)
        return jnp.pad(flat20, (0, _ROWS * _LANES - _CAND),
                       constant_values=pad_val).reshape(_ROWS, _LANES)

    s = tiles(probs[:, 1:_NUM_CLASSES], -1.0)
    tx1 = tiles(bx1[:, c0:c1:4], 0.0)
    ty1 = tiles(by1[:, c0:c1:4], 0.0)
    tx2 = tiles(bx2[:, c0:c1:4], 0.0)
    ty2 = tiles(by2[:, c0:c1:4], 0.0)

    det = _nms(s, tx1, ty1, tx2, ty2)
    det_boxes = jnp.stack([det[0, :_TOP_N], det[1, :_TOP_N],
                           det[2, :_TOP_N], det[3, :_TOP_N]], axis=1)
    det_labels = det[4, :_TOP_N].astype(jnp.int32)
    keep_scores = det[5, :_TOP_N]
    return det_boxes, det_labels, keep_scores


# speedup vs baseline: 1.7508x; 1.1788x over previous
import functools
import math

import jax
import jax.numpy as jnp
from jax import lax
from jax.experimental import pallas as pl
from jax.experimental.pallas import tpu as pltpu
from jax.experimental.pallas import tpu_sc as plsc

_NUM_CLASSES = 21
_POOL = 7
_STRIDE = 16.0
_SCORE_TH = 0.05
_NMS_TH = 0.5
_TOP_N = 100
_PRE_NMS = 2000
_MIN_SIZE = 1.0
_FEAT_HW = 50
_DW_CLAMP = math.log(1000.0 / 16.0)

_N = 1000
_C = 512
_CELLS = _FEAT_HW * _FEAT_HW
_CELLS_PAD = 2560
_P49 = _POOL * _POOL
_NP = _N * _P49
_NP_PAD = 49152
_H = 4096
_BJ6 = 2048
_NJ6 = _H // _BJ6
_BJ7 = 2048
_NJ7 = _H // _BJ7
_BK7 = 512
_NK7 = _H // _BK7
_CAND = _N * (_NUM_CLASSES - 1)
_ROWS = 160
_LANES = 128
_BIG = 1 << 30


def _tr_body(x_ref, o_ref):
    o_ref[...] = x_ref[...].T


def _transpose_feat(featp):
    return pl.pallas_call(
        _tr_body,
        grid=(_CELLS_PAD // 256,),
        in_specs=[pl.BlockSpec((_C, 256), lambda i: (0, i))],
        out_specs=pl.BlockSpec((256, _C), lambda i: (i, 0)),
        out_shape=jax.ShapeDtypeStruct((_CELLS_PAD, _C), jnp.float32),
    )(featp)


def _idx_body(p_ref, o_ref):
    pp = p_ref[...]
    x1 = pp[:, 0:1] / _STRIDE
    y1 = pp[:, 1:2] / _STRIDE
    x2 = pp[:, 2:3] / _STRIDE
    y2 = pp[:, 3:4] / _STRIDE
    col = lax.broadcasted_iota(jnp.int32, (_N, _LANES), 1)
    px = (col % _POOL).astype(jnp.float32)
    py = (col // _POOL).astype(jnp.float32)
    gx = (px + 0.5) / _POOL
    gy = (py + 0.5) / _POOL
    sx = x1 + gx * (x2 - x1)
    sy = y1 + gy * (y2 - y1)
    ix = jnp.clip(jnp.floor(sx), 0.0, _FEAT_HW - 1).astype(jnp.int32)
    iy = jnp.clip(jnp.floor(sy), 0.0, _FEAT_HW - 1).astype(jnp.int32)
    flat = iy * _FEAT_HW + ix
    o_ref[...] = jnp.where(col < _P49, flat, 0)


def _roi_idx(proposals):
    return pl.pallas_call(
        _idx_body,
        in_specs=[pl.BlockSpec((_N, 4), lambda: (0, 0))],
        out_specs=pl.BlockSpec((_N, _LANES), lambda: (0, 0)),
        out_shape=jax.ShapeDtypeStruct((_N, _LANES), jnp.int32),
    )(proposals)


_NW = 32
_BPW = _NP_PAD // _NW
_GCH = 64


def _sc_gather(table, idx1d):
    mesh = plsc.VectorSubcoreMesh(core_axis_name="core",
                                  subcore_axis_name="subcore")

    @functools.partial(
        pl.kernel,
        out_type=jax.ShapeDtypeStruct((_NP_PAD, _C), jnp.float32),
        scratch_types=[
            pltpu.VMEM((_BPW,), jnp.int32),
            pltpu.VMEM((_GCH, _C), jnp.float32),
            pltpu.SemaphoreType.DMA,
        ],
        mesh=mesh,
    )
    def k(x_hbm, i_hbm, o_hbm, idx_v, rows_v, sem):
        wid = lax.axis_index("subcore") * 2 + lax.axis_index("core")
        base = wid * _BPW
        pltpu.sync_copy(i_hbm.at[pl.ds(base, _BPW)], idx_v)

        @pl.loop(0, _BPW // _GCH)
        def _(c):
            pltpu.async_copy(
                x_hbm.at[idx_v.at[pl.ds(c * _GCH, _GCH)]], rows_v, sem
            ).wait()
            pltpu.sync_copy(rows_v, o_hbm.at[pl.ds(base + c * _GCH, _GCH)])

    return k(table, idx1d)


def _fc6_body(g_ref, b_ref, w_hbm, o_ref, wbuf, sem):
    p = pl.program_id(0)
    j = pl.program_id(1)
    step = p * _NJ6 + j
    nsteps = _P49 * _NJ6

    @pl.when(step == 0)
    def _():
        pltpu.make_async_copy(w_hbm.at[:, p, pl.ds(j * _BJ6, _BJ6)],
                              wbuf.at[0], sem.at[0]).start()

    @pl.when(step + 1 < nsteps)
    def _():
        jn = jnp.where(j + 1 < _NJ6, j + 1, 0)
        pn = jnp.where(j + 1 < _NJ6, p, p + 1)
        pltpu.make_async_copy(w_hbm.at[:, pn, pl.ds(jn * _BJ6, _BJ6)],
                              wbuf.at[(step + 1) % 2],
                              sem.at[(step + 1) % 2]).start()

    slot = step % 2
    pltpu.make_async_copy(w_hbm.at[:, p, pl.ds(j * _BJ6, _BJ6)],
                          wbuf.at[slot], sem.at[slot]).wait()
    partial = jnp.dot(g_ref[...], wbuf[slot],
                      preferred_element_type=jnp.float32)
    js = pl.ds(j * _BJ6, _BJ6)

    @pl.when(p == 0)
    def _():
        o_ref[:, js] = partial

    @pl.when(p > 0)
    def _():
        o_ref[:, js] += partial

    @pl.when(p == _P49 - 1)
    def _():
        o_ref[:, js] = jnp.maximum(o_ref[:, js] + b_ref[0:1, js], 0.0)


def _fc6(G, W6r, b6):
    return pl.pallas_call(
        _fc6_body,
        grid=(_P49, _NJ6),
        in_specs=[
            pl.BlockSpec((_N, _C), lambda p, j: (p, 0)),
            pl.BlockSpec((1, _H), lambda p, j: (0, 0)),
            pl.BlockSpec(memory_space=pl.ANY),
        ],
        out_specs=pl.BlockSpec((_N, _H), lambda p, j: (0, 0)),
        out_shape=jax.ShapeDtypeStruct((_N, _H), jnp.float32),
        scratch_shapes=[
            pltpu.VMEM((2, _C, _BJ6), jnp.float32),
            pltpu.SemaphoreType.DMA((2,)),
        ],
        compiler_params=pltpu.CompilerParams(
            dimension_semantics=("arbitrary", "arbitrary")),
    )(G, b6, W6r)


def _fc7_body(h_ref, w_ref, b_ref, o_ref):
    j = pl.program_id(0)
    k = pl.program_id(1)
    partial = jnp.dot(h_ref[:, pl.ds(k * _BK7, _BK7)], w_ref[...],
                      preferred_element_type=jnp.float32)
    js = pl.ds(j * _BJ7, _BJ7)

    @pl.when(k == 0)
    def _():
        o_ref[:, js] = partial

    @pl.when(k > 0)
    def _():
        o_ref[:, js] += partial

    @pl.when(k == _NK7 - 1)
    def _():
        o_ref[:, js] = jnp.maximum(o_ref[:, js] + b_ref[0:1, js], 0.0)


def _fc7(h6, W7, b7):
    return pl.pallas_call(
        _fc7_body,
        grid=(_NJ7, _NK7),
        in_specs=[
            pl.BlockSpec((_N, _H), lambda j, k: (0, 0)),
            pl.BlockSpec((_BK7, _BJ7), lambda j, k: (k, j)),
            pl.BlockSpec((1, _H), lambda j, k: (0, 0)),
        ],
        out_specs=pl.BlockSpec((_N, _H), lambda j, k: (0, 0)),
        out_shape=jax.ShapeDtypeStruct((_N, _H), jnp.float32),
        compiler_params=pltpu.CompilerParams(
            dimension_semantics=("arbitrary", "arbitrary")),
    )(h6, W7, b7)


def _decode_body(h_ref, w_ref, b_ref, p_ref, img_ref, s_ref,
                 x1_ref, y1_ref, x2_ref, y2_ref):
    logits = jnp.dot(h_ref[...], w_ref[...],
                     preferred_element_type=jnp.float32) + b_ref[0:1, :]
    col = lax.broadcasted_iota(jnp.int32, (_N, _LANES), 1)
    ninf = jnp.float32(-jnp.inf)
    clsmask = col < _NUM_CLASSES
    m = jnp.max(jnp.where(clsmask, logits, ninf), axis=1, keepdims=True)
    e = jnp.exp(logits - m)
    ssum = jnp.sum(jnp.where(clsmask, e, 0.0), axis=1, keepdims=True)
    s_ref[...] = e / ssum

    comp = (col - _NUM_CLASSES) % 4
    std = jnp.where(comp < 2, jnp.float32(0.1), jnp.float32(0.2))
    r0 = logits * std
    a0 = r0
    a1 = jnp.roll(r0, -1, axis=1)
    a2 = jnp.roll(r0, -2, axis=1)
    a3 = jnp.roll(r0, -3, axis=1)
    dw = jnp.minimum(a2, jnp.float32(_DW_CLAMP))
    dh = jnp.minimum(a3, jnp.float32(_DW_CLAMP))

    pp = p_ref[...]
    w = pp[:, 2:3] - pp[:, 0:1]
    h = pp[:, 3:4] - pp[:, 1:2]
    cx = pp[:, 0:1] + 0.5 * w
    cy = pp[:, 1:2] + 0.5 * h
    pcx = a0 * w + cx
    pcy = a1 * h + cy
    pw = jnp.exp(dw) * w
    ph = jnp.exp(dh) * h
    him = img_ref[0, 0]
    wim = img_ref[0, 1]
    x1_ref[...] = jnp.clip(pcx - 0.5 * pw, 0.0, wim)
    y1_ref[...] = jnp.clip(pcy - 0.5 * ph, 0.0, him)
    x2_ref[...] = jnp.clip(pcx + 0.5 * pw, 0.0, wim)
    y2_ref[...] = jnp.clip(pcy + 0.5 * ph, 0.0, him)


def _decode(h7, Wcat, bcat, proposals, imgf):
    outs = [jax.ShapeDtypeStruct((_N, _LANES), jnp.float32)] * 5
    return pl.pallas_call(
        _decode_body,
        in_specs=[
            pl.BlockSpec((_N, _H), lambda: (0, 0)),
            pl.BlockSpec((_H, _LANES), lambda: (0, 0)),
            pl.BlockSpec((1, _LANES), lambda: (0, 0)),
            pl.BlockSpec((_N, 4), lambda: (0, 0)),
            pl.BlockSpec((1, 2), lambda: (0, 0)),
        ],
        out_specs=[pl.BlockSpec((_N, _LANES), lambda: (0, 0))] * 5,
        out_shape=outs,
    )(h7, Wcat, bcat, proposals, imgf)


def _nms_body(s_ref, x1_ref, y1_ref, x2_ref, y2_ref, det_ref):
    ninf = jnp.float32(-jnp.inf)
    s = s_ref[...]
    x1 = x1_ref[...]
    y1 = y1_ref[...]
    x2 = x2_ref[...]
    y2 = y2_ref[...]
    fi = (lax.broadcasted_iota(jnp.int32, (_ROWS, _LANES), 0) * _LANES
          + lax.broadcasted_iota(jnp.int32, (_ROWS, _LANES), 1))

    valid = (s > _SCORE_TH) & (x2 - x1 >= _MIN_SIZE) & (y2 - y1 >= _MIN_SIZE)
    msc = jnp.where(valid, s, ninf)
    key = lax.bitcast_convert_type(msc, jnp.int32)

    def bs_val(_, lh):
        lo, hi = lh
        mid = lo + (hi - lo + 1) // 2
        cnt = jnp.sum((key >= mid).astype(jnp.int32))
        ok = cnt >= _PRE_NMS
        return (jnp.where(ok, mid, lo), jnp.where(ok, hi, mid - 1))

    lo0 = jnp.int32(1)
    hi0 = jnp.int32(0x40000000)
    kth, _ = lax.fori_loop(0, 31, bs_val, (lo0, hi0))

    k1 = jnp.sum((key > kth).astype(jnp.int32))
    need = _PRE_NMS - k1
    ties = key == kth

    def bs_idx(_, lh):
        lo, hi = lh
        mid = (lo + hi) // 2
        c = jnp.sum((ties & (fi <= mid)).astype(jnp.int32))
        ok = c >= need
        return (jnp.where(ok, lo, mid + 1), jnp.where(ok, mid, hi))

    mstar, _ = lax.fori_loop(0, 16, bs_idx,
                             (jnp.int32(0), jnp.int32(_ROWS * _LANES - 1)))

    sel = (key > kth) | (ties & (fi <= mstar))
    sc0 = jnp.where(sel, msc, ninf)

    lab_all = (fi % (_NUM_CLASSES - 1) + 1).astype(jnp.float32)
    offv = lab_all * 1000.0
    ox1 = x1 + offv
    oy1 = y1 + offv
    ox2 = x2 + offv
    oy2 = y2 + offv
    area = (ox2 - ox1) * (oy2 - oy1)

    m0 = jnp.max(sc0)
    i_fb = jnp.min(jnp.where(sc0 == m0, fi, _BIG))
    i_fb = jnp.minimum(i_fb, jnp.int32(_CAND - 1))

    zrow = jnp.zeros((1, _LANES), jnp.float32)
    ciota = lax.broadcasted_iota(jnp.int32, (1, _LANES), 1)

    def body(k, carry):
        sc, ax1, ay1, ax2, ay2, alab, asc = carry
        mk = jnp.max(sc)
        i = jnp.min(jnp.where(sc == mk, fi, _BIG))
        i = jnp.where(mk == ninf, i_fb, i)
        pick = fi == i
        bx1 = jnp.sum(jnp.where(pick, x1, 0.0))
        by1 = jnp.sum(jnp.where(pick, y1, 0.0))
        bx2 = jnp.sum(jnp.where(pick, x2, 0.0))
        by2 = jnp.sum(jnp.where(pick, y2, 0.0))
        labi = i % (_NUM_CLASSES - 1) + 1
        labf = labi.astype(jnp.float32)
        off = labf * 1000.0
        b0 = bx1 + off
        b1 = by1 + off
        b2 = bx2 + off
        b3 = by2 + off
        a_b = (b2 - b0) * (b3 - b1)
        xx1 = jnp.maximum(b0, ox1)
        yy1 = jnp.maximum(b1, oy1)
        xx2 = jnp.minimum(b2, ox2)
        yy2 = jnp.minimum(b3, oy2)
        inter = jnp.maximum(xx2 - xx1, 0.0) * jnp.maximum(yy2 - yy1, 0.0)
        iou = inter / (a_b + area - inter)
        sc = jnp.where(iou > _NMS_TH, ninf, sc)
        sc = jnp.where(pick, ninf, sc)
        cm = ciota == k
        ax1 = jnp.where(cm, bx1, ax1)
        ay1 = jnp.where(cm, by1, ay1)
        ax2 = jnp.where(cm, bx2, ax2)
        ay2 = jnp.where(cm, by2, ay2)
        alab = jnp.where(cm, labf, alab)
        asc = jnp.where(cm, mk, asc)
        return sc, ax1, ay1, ax2, ay2, alab, asc

    init = (sc0, zrow, zrow, zrow, zrow, zrow, zrow)
    _, ax1, ay1, ax2, ay2, alab, asc = lax.fori_loop(0, _TOP_N, body, init)
    det_ref[...] = jnp.concatenate(
        [ax1, ay1, ax2, ay2, alab, asc, zrow, zrow], axis=0)


def _nms(s, x1, y1, x2, y2):
    return pl.pallas_call(
        _nms_body,
        in_specs=[pl.BlockSpec((_ROWS, _LANES), lambda: (0, 0))] * 5,
        out_specs=pl.BlockSpec((8, _LANES), lambda: (0, 0)),
        out_shape=jax.ShapeDtypeStruct((8, _LANES), jnp.float32),
    )(s, x1, y1, x2, y2)


def kernel(feature_map, proposals, image_shape, W_fc6, b_fc6, W_fc7, b_fc7,
           W_cls, b_cls, W_reg, b_reg):
    feat = feature_map[0].reshape(_C, _CELLS)
    featp = jnp.pad(feat, ((0, 0), (0, _CELLS_PAD - _CELLS)))
    table = _transpose_feat(featp)

    flat = _roi_idx(proposals)
    idxf = flat[:, :_P49].T.reshape(_NP)
    idxp = jnp.pad(idxf, (0, _NP_PAD - _NP))
    G = _sc_gather(table, idxp)

    W6r = W_fc6.reshape(_C, _P49, _H)
    h6 = _fc6(G, W6r, b_fc6.reshape(1, _H))
    h7 = _fc7(h6, W_fc7, b_fc7.reshape(1, _H))

    npad = _LANES - (_NUM_CLASSES + 4 * _NUM_CLASSES)
    Wcat = jnp.pad(jnp.concatenate([W_cls, W_reg], axis=1),
                   ((0, 0), (0, npad)))
    bcat = jnp.pad(jnp.concatenate([b_cls, b_reg]), (0, npad)).reshape(1, _LANES)
    imgf = image_shape.astype(jnp.float32).reshape(1, 2)
    probs, bx1, by1, bx2, by2 = _decode(h7, Wcat, bcat, proposals, imgf)

    c0 = _NUM_CLASSES + 4
    c1 = _NUM_CLASSES + 4 * _NUM_CLASSES

    def tiles(a, pad_val):
        flat20 = a.reshape(_CAND)
        return jnp.pad(flat20, (0, _ROWS * _LANES - _CAND),
                       constant_values=pad_val).reshape(_ROWS, _LANES)

    s = tiles(probs[:, 1:_NUM_CLASSES], -1.0)
    tx1 = tiles(bx1[:, c0:c1:4], 0.0)
    ty1 = tiles(by1[:, c0:c1:4], 0.0)
    tx2 = tiles(bx2[:, c0:c1:4], 0.0)
    ty2 = tiles(by2[:, c0:c1:4], 0.0)

    det = _nms(s, tx1, ty1, tx2, ty2)
    det_boxes = jnp.stack([det[0, :_TOP_N], det[1, :_TOP_N],
                           det[2, :_TOP_N], det[3, :_TOP_N]], axis=1)
    det_labels = det[4, :_TOP_N].astype(jnp.int32)
    keep_scores = det[5, :_TOP_N]
    return det_boxes, det_labels, keep_scores
